# baseline (speedup 1.0000x reference)
import functools

import jax
import jax.numpy as jnp
from jax import lax
from jax.experimental import pallas as pl
from jax.experimental.pallas import tpu as pltpu
from jax.experimental.pallas import tpu_sc as plsc

_N = 10000
_E = 320000
_D = 128
_EPS = 1e-16
_NW = 32
_EW = _E // _NW
_B = 128
_NB = 80
_EWP = _NB * _B

_mesh = plsc.VectorSubcoreMesh(core_axis_name="c", subcore_axis_name="s")


def _dense_body(x_ref, w_ref, avs_ref, avd_ref, h_ref, as_ref, ad_ref, m_ref):
    h = jnp.dot(x_ref[...], w_ref[...], preferred_element_type=jnp.float32)
    h_ref[...] = h
    a_s = jnp.dot(h, avs_ref[...], preferred_element_type=jnp.float32)
    a_d = jnp.dot(h, avd_ref[...], preferred_element_type=jnp.float32)
    as_ref[...] = a_s
    ad_ref[...] = a_d
    m = jnp.maximum(jnp.max(a_s) + jnp.max(a_d), 0.0)
    m_ref[...] = jnp.full((16,), m, jnp.float32)


def _dense(x, w, att_s, att_d):
    return pl.pallas_call(
        _dense_body,
        out_shape=(
            jax.ShapeDtypeStruct((_N, _D), jnp.float32),
            jax.ShapeDtypeStruct((_N,), jnp.float32),
            jax.ShapeDtypeStruct((_N,), jnp.float32),
            jax.ShapeDtypeStruct((16,), jnp.float32),
        ),
    )(x, w, att_s, att_d)


def _mid_body(g_ref, s1p_ref, b1_ref, w2_ref, avs_ref, avd_ref,
              h2_ref, as_ref, ad_ref, m_ref):
    g = g_ref[0] + g_ref[1]
    s1 = jnp.sum(s1p_ref[...], axis=0)
    out1 = g / (s1[:, None] + _EPS) + b1_ref[...][None, :]
    x2 = jnp.maximum(out1, 0.0)
    h2 = jnp.dot(x2, w2_ref[...], preferred_element_type=jnp.float32)
    h2_ref[...] = h2
    a_s = jnp.dot(h2, avs_ref[...], preferred_element_type=jnp.float32)
    a_d = jnp.dot(h2, avd_ref[...], preferred_element_type=jnp.float32)
    as_ref[...] = a_s
    ad_ref[...] = a_d
    m = jnp.maximum(jnp.max(a_s) + jnp.max(a_d), 0.0)
    m_ref[...] = jnp.full((16,), m, jnp.float32)


def _mid(g, s1p, b1, w2, att_s, att_d):
    return pl.pallas_call(
        _mid_body,
        out_shape=(
            jax.ShapeDtypeStruct((_N, _D), jnp.float32),
            jax.ShapeDtypeStruct((_N,), jnp.float32),
            jax.ShapeDtypeStruct((_N,), jnp.float32),
            jax.ShapeDtypeStruct((16,), jnp.float32),
        ),
    )(g, s1p, b1, w2, att_s, att_d)


def _colsum_body(p_ref, o_ref):
    o_ref[...] = jnp.sum(p_ref[...], axis=0)


def _colsum(p):
    return pl.pallas_call(
        _colsum_body,
        out_shape=jax.ShapeDtypeStruct((_N,), jnp.float32),
    )(p)


def _final_body(wp_ref, h2_ref, b2_ref, o_ref):
    w = jnp.sum(wp_ref[...], axis=0)
    o_ref[...] = (jnp.dot(w, h2_ref[...], preferred_element_type=jnp.float32)
                  * (1.0 / _N) + b2_ref[...])


def _final(wp, h2, b2):
    return pl.pallas_call(
        _final_body,
        out_shape=jax.ShapeDtypeStruct((_D,), jnp.float32),
    )(wp, h2, b2)


def _edges_body(srcg, dstg, as_hbm, ad_hbm, m_hbm,
                ex_out, sp_out,
                srcd, dstd, asv, adv, spv, exd, mv):
    c = lax.axis_index("c")
    s = lax.axis_index("s")
    wid = c * 16 + s
    pltpu.sync_copy(srcg.at[wid], srcd)
    pltpu.sync_copy(dstg.at[wid], dstd)
    pltpu.sync_copy(as_hbm, asv)
    pltpu.sync_copy(ad_hbm, adv)
    pltpu.sync_copy(m_hbm, mv)
    mvec = mv[...]
    z16 = jnp.zeros((16,), jnp.float32)

    def zero(i, _):
        spv[pl.ds(i * 16, 16)] = z16
        return 0
    lax.fori_loop(0, _N // 16, zero, 0)
    for k in range(15):
        exd[pl.ds(_EW + k * 16, 16)] = z16

    def group(b, j):
        s16 = srcd[b, pl.ds(j * 16, 16)]
        d16 = dstd[b, pl.ds(j * 16, 16)]
        e = plsc.load_gather(asv, [s16]) + plsc.load_gather(adv, [d16])
        e = jnp.maximum(e, 0.2 * e)
        ex = jnp.exp(e - mvec)
        exd[pl.ds(b * _B + j * 16, 16)] = ex
        plsc.addupdate_scatter(spv, [d16], ex)

    def body(b, _):
        for j in range(_B // 16):
            group(b, j)
        return 0
    lax.fori_loop(0, 78, body, 0)
    group(78, 0)
    pltpu.sync_copy(exd, ex_out.at[wid, 0])
    pltpu.sync_copy(spv, sp_out.at[wid, 0])


_edges = functools.partial(
    pl.kernel,
    out_type=(
        jax.ShapeDtypeStruct((_NW, 1, _EWP), jnp.float32),
        jax.ShapeDtypeStruct((_NW, 1, _N), jnp.float32),
    ),
    mesh=_mesh,
    compiler_params=pltpu.CompilerParams(needs_layout_passes=False),
    scratch_types=[
        pltpu.VMEM((_NB, _B), jnp.int32),
        pltpu.VMEM((_NB, _B), jnp.int32),
        pltpu.VMEM((_N,), jnp.float32),
        pltpu.VMEM((_N,), jnp.float32),
        pltpu.VMEM((_N,), jnp.float32),
        pltpu.VMEM((_EWP,), jnp.float32),
        pltpu.VMEM((16,), jnp.float32),
    ],
)(_edges_body)


_NBM = 79


def _msg_phase(b, h1_hbm, mef, dstd, gsp, wid, me, other):
    rows_p, meb_p, sem_g_p, sem_s_p, sem_m_p = me
    rows_o, meb_o, sem_g_o, sem_s_o, sem_m_o = other

    @pl.when(b + 1 < _NBM)
    def _prefetch_gather():
        @pl.when(b >= 1)
        def _drain_prev_scatter():
            pltpu.make_async_copy(
                rows_o, gsp.at[dstd.at[b]], sem_s_o).wait()
        pltpu.make_async_copy(
            mef.at[wid, 0, pl.ds(b * 2 * _B, 2 * _B)], meb_o, sem_m_o).wait()
        pltpu.async_copy(h1_hbm.at[meb_o.at[pl.ds(0, _B)]], rows_o, sem_g_o)

    pltpu.make_async_copy(
        h1_hbm.at[meb_p.at[pl.ds(0, _B)]], rows_p, sem_g_p).wait()

    def scale_group(g, _):
        exv16 = plsc.bitcast(meb_p[pl.ds(_B + g * 16, 16)], jnp.float32)
        for r in range(16):
            exv = jnp.full((16,), exv16[r], jnp.float32)
            row = g * 16 + r
            for j in range(8):
                rows_p[row, pl.ds(j * 16, 16)] = (
                    rows_p[row, pl.ds(j * 16, 16)] * exv)
        return 0
    lax.fori_loop(0, _B // 16, scale_group, 0)
    pltpu.async_copy(rows_p, gsp.at[dstd.at[b]], sem_s_p, add=True)

    @pl.when(b + 2 < _NBM)
    def _prefetch_meta():
        pltpu.async_copy(mef.at[wid, 0, pl.ds((b + 2) * 2 * _B, 2 * _B)],
                         meb_p, sem_m_p)


def _msg_body(h1_hbm, mef, dstg, zero_hbm, g_out,
              dstd, meb0, meb1, rows0, rows1,
              sem_g0, sem_g1, sem_s0, sem_s1, sem_m0, sem_m1, gsp):
    c = lax.axis_index("c")
    s = lax.axis_index("s")
    wid = c * 16 + s
    pltpu.sync_copy(dstg.at[wid], dstd)

    row0 = pl.multiple_of(s * 632, 8)

    @pl.when(s < 15)
    def _zero_main():
        pltpu.sync_copy(zero_hbm.at[pl.ds(row0, 632)],
                        gsp.at[pl.ds(row0, 632)])

    @pl.when(s == 15)
    def _zero_tail():
        pltpu.sync_copy(zero_hbm.at[pl.ds(9480, 520)],
                        gsp.at[pl.ds(9480, 520)])

    plsc.subcore_barrier()

    set0 = (rows0, meb0, sem_g0, sem_s0, sem_m0)
    set1 = (rows1, meb1, sem_g1, sem_s1, sem_m1)

    pltpu.sync_copy(mef.at[wid, 0, pl.ds(0, 2 * _B)], meb0)
    pltpu.async_copy(h1_hbm.at[meb0.at[pl.ds(0, _B)]], rows0, sem_g0)
    pltpu.async_copy(mef.at[wid, 0, pl.ds(2 * _B, 2 * _B)], meb1, sem_m1)

    def body(b, _):
        parity = lax.rem(b, 2)

        @pl.when(parity == 0)
        def _even():
            _msg_phase(b, h1_hbm, mef, dstd, gsp, wid, set0, set1)

        @pl.when(parity == 1)
        def _odd():
            _msg_phase(b, h1_hbm, mef, dstd, gsp, wid, set1, set0)
        return 0
    lax.fori_loop(0, _NBM, body, 0)

    pltpu.make_async_copy(rows1, gsp.at[dstd.at[77]], sem_s1).wait()
    pltpu.make_async_copy(rows0, gsp.at[dstd.at[78]], sem_s0).wait()
    plsc.subcore_barrier()

    @pl.when(s < 15)
    def _dump_main():
        pltpu.sync_copy(gsp.at[pl.ds(row0, 632)],
                        g_out.at[c, pl.ds(row0, 632)])

    @pl.when(s == 15)
    def _dump_tail():
        pltpu.sync_copy(gsp.at[pl.ds(9480, 520)],
                        g_out.at[c, pl.ds(9480, 520)])


_msg = functools.partial(
    pl.kernel,
    out_type=jax.ShapeDtypeStruct((2, _N, _D), jnp.float32),
    mesh=_mesh,
    compiler_params=pltpu.CompilerParams(needs_layout_passes=False),
    scratch_types=[
        pltpu.VMEM((_NB, _B), jnp.int32),
        pltpu.VMEM((2 * _B,), jnp.int32),
        pltpu.VMEM((2 * _B,), jnp.int32),
        pltpu.VMEM((_B, _D), jnp.float32),
        pltpu.VMEM((_B, _D), jnp.float32),
        pltpu.SemaphoreType.DMA,
        pltpu.SemaphoreType.DMA,
        pltpu.SemaphoreType.DMA,
        pltpu.SemaphoreType.DMA,
        pltpu.SemaphoreType.DMA,
        pltpu.SemaphoreType.DMA,
        pltpu.VMEM_SHARED((_N, _D), jnp.float32),
    ],
)(_msg_body)


def _wpass_body(srcg, dstg, exg, s2_hbm, wp_out,
                srcd, dstd, exd, s2v, wpv):
    c = lax.axis_index("c")
    s = lax.axis_index("s")
    wid = c * 16 + s
    pltpu.sync_copy(srcg.at[wid], srcd)
    pltpu.sync_copy(dstg.at[wid], dstd)
    pltpu.sync_copy(exg.at[wid], exd)
    pltpu.sync_copy(s2_hbm, s2v)
    z16 = jnp.zeros((16,), jnp.float32)

    def zero(i, _):
        wpv[pl.ds(i * 16, 16)] = z16
        return 0
    lax.fori_loop(0, _N // 16, zero, 0)

    def group(b, j):
        s16 = srcd[b, pl.ds(j * 16, 16)]
        d16 = dstd[b, pl.ds(j * 16, 16)]
        ex = exd[b, pl.ds(j * 16, 16)]
        sv = plsc.load_gather(s2v, [d16])
        attn = ex / (sv + _EPS)
        plsc.addupdate_scatter(wpv, [s16], attn)

    def body(b, _):
        for j in range(_B // 16):
            group(b, j)
        return 0
    lax.fori_loop(0, 78, body, 0)
    group(78, 0)
    pltpu.sync_copy(wpv, wp_out.at[wid, 0])


_wpass = functools.partial(
    pl.kernel,
    out_type=jax.ShapeDtypeStruct((_NW, 1, _N), jnp.float32),
    mesh=_mesh,
    compiler_params=pltpu.CompilerParams(needs_layout_passes=False),
    scratch_types=[
        pltpu.VMEM((_NB, _B), jnp.int32),
        pltpu.VMEM((_NB, _B), jnp.int32),
        pltpu.VMEM((_NB, _B), jnp.float32),
        pltpu.VMEM((_N,), jnp.float32),
        pltpu.VMEM((_N,), jnp.float32),
    ],
)(_wpass_body)


def kernel(x, edge_index, W1, att_src1, att_dst1, b1,
           W2, att_src2, att_dst2, b2):
    pad = jnp.zeros((_NW, _EWP - _EW), jnp.int32)
    srcg = jnp.concatenate(
        [edge_index[0].reshape(_NW, _EW), pad], axis=1).reshape(_NW, _NB, _B)
    dstg = jnp.concatenate(
        [edge_index[1].reshape(_NW, _EW), pad], axis=1).reshape(_NW, _NB, _B)
    zeros = jnp.zeros((_N, _D), jnp.float32)

    h1, as1, ad1, m1v = _dense(x, W1, att_src1, att_dst1)
    ex1, s1p = _edges(srcg, dstg, as1, ad1, m1v)
    mef = jnp.concatenate(
        [srcg, jax.lax.bitcast_convert_type(ex1.reshape(_NW, _NB, _B),
                                            jnp.int32)],
        axis=2).reshape(_NW, 1, 2 * _EWP)
    g = _msg(h1, mef, dstg, zeros)
    h2, as2, ad2, m2v = _mid(g, s1p.reshape(_NW, _N), b1, W2,
                             att_src2, att_dst2)
    ex2, s2p = _edges(srcg, dstg, as2, ad2, m2v)
    s2 = _colsum(s2p.reshape(_NW, _N))
    wp = _wpass(srcg, dstg, ex2.reshape(_NW, _NB, _B), s2)
    return _final(wp.reshape(_NW, _N), h2, b2)


# speedup vs baseline: 1.0147x; 1.0147x over previous
import functools

import jax
import jax.numpy as jnp
from jax import lax
from jax.experimental import pallas as pl
from jax.experimental.pallas import tpu as pltpu
from jax.experimental.pallas import tpu_sc as plsc

_N = 10000
_E = 320000
_D = 128
_EPS = 1e-16
_NW = 32
_EW = _E // _NW
_B = 128
_NB = 80
_EWP = _NB * _B

_mesh = plsc.VectorSubcoreMesh(core_axis_name="c", subcore_axis_name="s")


def _dense_body(x_ref, w_ref, avs_ref, avd_ref, h_ref, as_ref, ad_ref, m_ref):
    h = jnp.dot(x_ref[...], w_ref[...], preferred_element_type=jnp.float32)
    h_ref[...] = h
    a_s = jnp.dot(h, avs_ref[...], preferred_element_type=jnp.float32)
    a_d = jnp.dot(h, avd_ref[...], preferred_element_type=jnp.float32)
    as_ref[...] = a_s
    ad_ref[...] = a_d
    m = jnp.maximum(jnp.max(a_s) + jnp.max(a_d), 0.0)
    m_ref[...] = jnp.full((16,), m, jnp.float32)


def _dense(x, w, att_s, att_d):
    return pl.pallas_call(
        _dense_body,
        out_shape=(
            jax.ShapeDtypeStruct((_N, _D), jnp.float32),
            jax.ShapeDtypeStruct((_N,), jnp.float32),
            jax.ShapeDtypeStruct((_N,), jnp.float32),
            jax.ShapeDtypeStruct((16,), jnp.float32),
        ),
    )(x, w, att_s, att_d)


def _mid_body(g_ref, s1p_ref, b1_ref, w2_ref, avs_ref, avd_ref,
              h2_ref, as_ref, ad_ref, m_ref):
    g = g_ref[0] + g_ref[1]
    s1 = jnp.sum(s1p_ref[...], axis=0)
    out1 = g / (s1[:, None] + _EPS) + b1_ref[...][None, :]
    x2 = jnp.maximum(out1, 0.0)
    h2 = jnp.dot(x2, w2_ref[...], preferred_element_type=jnp.float32)
    h2_ref[...] = h2
    a_s = jnp.dot(h2, avs_ref[...], preferred_element_type=jnp.float32)
    a_d = jnp.dot(h2, avd_ref[...], preferred_element_type=jnp.float32)
    as_ref[...] = a_s
    ad_ref[...] = a_d
    m = jnp.maximum(jnp.max(a_s) + jnp.max(a_d), 0.0)
    m_ref[...] = jnp.full((16,), m, jnp.float32)


def _mid(g, s1p, b1, w2, att_s, att_d):
    return pl.pallas_call(
        _mid_body,
        out_shape=(
            jax.ShapeDtypeStruct((_N, _D), jnp.float32),
            jax.ShapeDtypeStruct((_N,), jnp.float32),
            jax.ShapeDtypeStruct((_N,), jnp.float32),
            jax.ShapeDtypeStruct((16,), jnp.float32),
        ),
    )(g, s1p, b1, w2, att_s, att_d)


def _colsum_body(p_ref, o_ref):
    o_ref[...] = jnp.sum(p_ref[...], axis=0)


def _colsum(p):
    return pl.pallas_call(
        _colsum_body,
        out_shape=jax.ShapeDtypeStruct((_N,), jnp.float32),
    )(p)


def _final_body(wp_ref, h2_ref, b2_ref, o_ref):
    w = jnp.sum(wp_ref[...], axis=0)
    o_ref[...] = (jnp.dot(w, h2_ref[...], preferred_element_type=jnp.float32)
                  * (1.0 / _N) + b2_ref[...])


def _final(wp, h2, b2):
    return pl.pallas_call(
        _final_body,
        out_shape=jax.ShapeDtypeStruct((_D,), jnp.float32),
    )(wp, h2, b2)


def _edges_body(srcg, dstg, as_hbm, ad_hbm, m_hbm,
                ex_out, sp_out,
                srcd, dstd, asv, adv, spv, exd, mv):
    c = lax.axis_index("c")
    s = lax.axis_index("s")
    wid = c * 16 + s
    pltpu.sync_copy(srcg.at[wid], srcd)
    pltpu.sync_copy(dstg.at[wid], dstd)
    pltpu.sync_copy(as_hbm, asv)
    pltpu.sync_copy(ad_hbm, adv)
    pltpu.sync_copy(m_hbm, mv)
    mvec = mv[...]
    z16 = jnp.zeros((16,), jnp.float32)

    def zero(i, _):
        spv[pl.ds(i * 16, 16)] = z16
        return 0
    lax.fori_loop(0, _N // 16, zero, 0)
    for k in range(15):
        exd[pl.ds(_EW + k * 16, 16)] = z16

    def group(b, j):
        s16 = srcd[b, pl.ds(j * 16, 16)]
        d16 = dstd[b, pl.ds(j * 16, 16)]
        e = plsc.load_gather(asv, [s16]) + plsc.load_gather(adv, [d16])
        e = jnp.maximum(e, 0.2 * e)
        ex = jnp.exp(e - mvec)
        exd[pl.ds(b * _B + j * 16, 16)] = ex
        plsc.addupdate_scatter(spv, [d16], ex)

    def body(b, _):
        for j in range(_B // 16):
            group(b, j)
        return 0
    lax.fori_loop(0, 78, body, 0)
    group(78, 0)
    pltpu.sync_copy(exd, ex_out.at[wid, 0])
    pltpu.sync_copy(spv, sp_out.at[wid, 0])


_edges = functools.partial(
    pl.kernel,
    out_type=(
        jax.ShapeDtypeStruct((_NW, 1, _EWP), jnp.float32),
        jax.ShapeDtypeStruct((_NW, 1, _N), jnp.float32),
    ),
    mesh=_mesh,
    compiler_params=pltpu.CompilerParams(needs_layout_passes=False),
    scratch_types=[
        pltpu.VMEM((_NB, _B), jnp.int32),
        pltpu.VMEM((_NB, _B), jnp.int32),
        pltpu.VMEM((_N,), jnp.float32),
        pltpu.VMEM((_N,), jnp.float32),
        pltpu.VMEM((_N,), jnp.float32),
        pltpu.VMEM((_EWP,), jnp.float32),
        pltpu.VMEM((16,), jnp.float32),
    ],
)(_edges_body)


_NBM = 79


def _msg_phase(b, h1_hbm, srcf, exf, dstd, gsp, wid, me, other):
    rows_p, srcb_p, exb_p, sem_g_p, sem_s_p, sem_m_p = me
    rows_o, srcb_o, exb_o, sem_g_o, sem_s_o, sem_m_o = other

    @pl.when(b + 1 < _NBM)
    def _prefetch_gather():
        @pl.when(b >= 1)
        def _drain_prev_scatter():
            pltpu.make_async_copy(
                rows_o, gsp.at[dstd.at[b]], sem_s_o).wait()
        pltpu.make_async_copy(
            srcf.at[wid, 0, pl.ds(b * _B, _B)], srcb_o, sem_m_o).wait()
        pltpu.make_async_copy(
            exf.at[wid, 0, pl.ds(b * _B, _B)], exb_o, sem_m_o).wait()
        pltpu.async_copy(h1_hbm.at[srcb_o], rows_o, sem_g_o)

    pltpu.make_async_copy(h1_hbm.at[srcb_p], rows_p, sem_g_p).wait()

    def scale_group(g, _):
        exv16 = exb_p[pl.ds(g * 16, 16)]
        for r in range(16):
            exv = jnp.full((16,), exv16[r], jnp.float32)
            row = g * 16 + r
            for j in range(8):
                rows_p[row, pl.ds(j * 16, 16)] = (
                    rows_p[row, pl.ds(j * 16, 16)] * exv)
        return 0
    lax.fori_loop(0, _B // 16, scale_group, 0)
    pltpu.async_copy(rows_p, gsp.at[dstd.at[b]], sem_s_p, add=True)

    @pl.when(b + 2 < _NBM)
    def _prefetch_meta():
        pltpu.async_copy(srcf.at[wid, 0, pl.ds((b + 2) * _B, _B)],
                         srcb_p, sem_m_p)
        pltpu.async_copy(exf.at[wid, 0, pl.ds((b + 2) * _B, _B)],
                         exb_p, sem_m_p)


def _msg_body(h1_hbm, srcf, dstg, exf, zero_hbm, g_out,
              dstd, srcb0, srcb1, exb0, exb1, rows0, rows1,
              sem_g0, sem_g1, sem_s0, sem_s1, sem_m0, sem_m1, gsp):
    c = lax.axis_index("c")
    s = lax.axis_index("s")
    wid = c * 16 + s
    pltpu.sync_copy(dstg.at[wid], dstd)

    row0 = pl.multiple_of(s * 632, 8)

    @pl.when(s < 15)
    def _zero_main():
        pltpu.sync_copy(zero_hbm.at[pl.ds(row0, 632)],
                        gsp.at[pl.ds(row0, 632)])

    @pl.when(s == 15)
    def _zero_tail():
        pltpu.sync_copy(zero_hbm.at[pl.ds(9480, 520)],
                        gsp.at[pl.ds(9480, 520)])

    plsc.subcore_barrier()

    set0 = (rows0, srcb0, exb0, sem_g0, sem_s0, sem_m0)
    set1 = (rows1, srcb1, exb1, sem_g1, sem_s1, sem_m1)

    pltpu.sync_copy(srcf.at[wid, 0, pl.ds(0, _B)], srcb0)
    pltpu.sync_copy(exf.at[wid, 0, pl.ds(0, _B)], exb0)
    pltpu.async_copy(h1_hbm.at[srcb0], rows0, sem_g0)
    pltpu.async_copy(srcf.at[wid, 0, pl.ds(_B, _B)], srcb1, sem_m1)
    pltpu.async_copy(exf.at[wid, 0, pl.ds(_B, _B)], exb1, sem_m1)

    def body(b, _):
        parity = lax.rem(b, 2)

        @pl.when(parity == 0)
        def _even():
            _msg_phase(b, h1_hbm, srcf, exf, dstd, gsp, wid, set0, set1)

        @pl.when(parity == 1)
        def _odd():
            _msg_phase(b, h1_hbm, srcf, exf, dstd, gsp, wid, set1, set0)
        return 0
    lax.fori_loop(0, _NBM, body, 0)

    pltpu.make_async_copy(rows1, gsp.at[dstd.at[77]], sem_s1).wait()
    pltpu.make_async_copy(rows0, gsp.at[dstd.at[78]], sem_s0).wait()
    plsc.subcore_barrier()

    @pl.when(s < 15)
    def _dump_main():
        pltpu.sync_copy(gsp.at[pl.ds(row0, 632)],
                        g_out.at[c, pl.ds(row0, 632)])

    @pl.when(s == 15)
    def _dump_tail():
        pltpu.sync_copy(gsp.at[pl.ds(9480, 520)],
                        g_out.at[c, pl.ds(9480, 520)])


_msg = functools.partial(
    pl.kernel,
    out_type=jax.ShapeDtypeStruct((2, _N, _D), jnp.float32),
    mesh=_mesh,
    compiler_params=pltpu.CompilerParams(needs_layout_passes=False),
    scratch_types=[
        pltpu.VMEM((_NB, _B), jnp.int32),
        pltpu.VMEM((_B,), jnp.int32),
        pltpu.VMEM((_B,), jnp.int32),
        pltpu.VMEM((_B,), jnp.float32),
        pltpu.VMEM((_B,), jnp.float32),
        pltpu.VMEM((_B, _D), jnp.float32),
        pltpu.VMEM((_B, _D), jnp.float32),
        pltpu.SemaphoreType.DMA,
        pltpu.SemaphoreType.DMA,
        pltpu.SemaphoreType.DMA,
        pltpu.SemaphoreType.DMA,
        pltpu.SemaphoreType.DMA,
        pltpu.SemaphoreType.DMA,
        pltpu.VMEM_SHARED((_N, _D), jnp.float32),
    ],
)(_msg_body)


def _wpass_body(srcg, dstg, exg, s2_hbm, wp_out,
                srcd, dstd, exd, s2v, wpv):
    c = lax.axis_index("c")
    s = lax.axis_index("s")
    wid = c * 16 + s
    pltpu.sync_copy(srcg.at[wid], srcd)
    pltpu.sync_copy(dstg.at[wid], dstd)
    pltpu.sync_copy(exg.at[wid], exd)
    pltpu.sync_copy(s2_hbm, s2v)
    z16 = jnp.zeros((16,), jnp.float32)

    def zero(i, _):
        wpv[pl.ds(i * 16, 16)] = z16
        return 0
    lax.fori_loop(0, _N // 16, zero, 0)

    def group(b, j):
        s16 = srcd[b, pl.ds(j * 16, 16)]
        d16 = dstd[b, pl.ds(j * 16, 16)]
        ex = exd[b, pl.ds(j * 16, 16)]
        sv = plsc.load_gather(s2v, [d16])
        attn = ex / (sv + _EPS)
        plsc.addupdate_scatter(wpv, [s16], attn)

    def body(b, _):
        for j in range(_B // 16):
            group(b, j)
        return 0
    lax.fori_loop(0, 78, body, 0)
    group(78, 0)
    pltpu.sync_copy(wpv, wp_out.at[wid, 0])


_wpass = functools.partial(
    pl.kernel,
    out_type=jax.ShapeDtypeStruct((_NW, 1, _N), jnp.float32),
    mesh=_mesh,
    compiler_params=pltpu.CompilerParams(needs_layout_passes=False),
    scratch_types=[
        pltpu.VMEM((_NB, _B), jnp.int32),
        pltpu.VMEM((_NB, _B), jnp.int32),
        pltpu.VMEM((_NB, _B), jnp.float32),
        pltpu.VMEM((_N,), jnp.float32),
        pltpu.VMEM((_N,), jnp.float32),
    ],
)(_wpass_body)


def kernel(x, edge_index, W1, att_src1, att_dst1, b1,
           W2, att_src2, att_dst2, b2):
    pad = jnp.zeros((_NW, _EWP - _EW), jnp.int32)
    srcg = jnp.concatenate(
        [edge_index[0].reshape(_NW, _EW), pad], axis=1).reshape(_NW, _NB, _B)
    dstg = jnp.concatenate(
        [edge_index[1].reshape(_NW, _EW), pad], axis=1).reshape(_NW, _NB, _B)
    zeros = jnp.zeros((_N, _D), jnp.float32)

    h1, as1, ad1, m1v = _dense(x, W1, att_src1, att_dst1)
    ex1, s1p = _edges(srcg, dstg, as1, ad1, m1v)
    g = _msg(h1, srcg.reshape(_NW, 1, _EWP), dstg, ex1, zeros)
    h2, as2, ad2, m2v = _mid(g, s1p.reshape(_NW, _N), b1, W2,
                             att_src2, att_dst2)
    ex2, s2p = _edges(srcg, dstg, as2, ad2, m2v)
    s2 = _colsum(s2p.reshape(_NW, _N))
    wp = _wpass(srcg, dstg, ex2.reshape(_NW, _NB, _B), s2)
    return _final(wp.reshape(_NW, _N), h2, b2)
